# Initial kernel scaffold; baseline (speedup 1.0000x reference)
import functools

import jax
import jax.numpy as jnp
from jax import lax
from jax.experimental import pallas as pl
from jax.experimental.pallas import tpu as pltpu
from jax.experimental.pallas import tpu_sc as plsc

B, F, V, D = 4, 100000, 50000, 128
L = 16
C = 128
TOTAL = B * F
NCHUNK = TOTAL // C
NC, NS = 2, 16
NW = NC * NS


def _sc_gather(fs_flat, x_flat):
    mesh = plsc.VectorSubcoreMesh(core_axis_name="c", subcore_axis_name="s")

    @functools.partial(
        pl.kernel,
        mesh=mesh,
        out_type=jax.ShapeDtypeStruct((TOTAL, D), jnp.float32),
        scratch_types=[
            pltpu.VMEM((C,), jnp.int32),
            pltpu.VMEM((C, D), jnp.float32),
            pltpu.SemaphoreType.DMA,
        ],
    )
    def k(fs_hbm, x_hbm, out_hbm, idx_v, rows_v, sem):
        wid = lax.axis_index("s") * NC + lax.axis_index("c")
        base = NCHUNK // NW
        extra = NCHUNK - base * NW
        n = base + jnp.where(wid < extra, 1, 0).astype(jnp.int32)

        def body(i, carry):
            c = wid + i * NW
            start = c * C
            pltpu.sync_copy(fs_hbm.at[pl.ds(start, C)], idx_v)
            for kk in range(C // L):
                pos = start + kk * L + lax.iota(jnp.int32, L)
                boff = (pos // F) * V
                idx_v[pl.ds(kk * L, L)] = idx_v[pl.ds(kk * L, L)] + boff
            pltpu.async_copy(x_hbm.at[idx_v], rows_v, sem).wait()
            pltpu.sync_copy(rows_v, out_hbm.at[pl.ds(start, C)])
            return carry

        lax.fori_loop(0, n, body, 0)

    return k(fs_flat, x_flat)


def kernel(fs, x):
    fs_flat = fs.reshape(TOTAL).astype(jnp.int32)
    x_flat = x.reshape(B * V, D)
    out = _sc_gather(fs_flat, x_flat)
    return out.reshape(B, F, D)


# speedup vs baseline: 10.6397x; 10.6397x over previous
import functools

import jax
import jax.numpy as jnp
from jax import lax
from jax.experimental import pallas as pl
from jax.experimental.pallas import tpu as pltpu
from jax.experimental.pallas import tpu_sc as plsc

B, F, V, D = 4, 100000, 50000, 128
C = 80
TOTAL = B * F
NCHUNK = TOTAL // C
CPB = F // C
NC, NS = 2, 16
NW = NC * NS


def _sc_gather(fs_flat, x):
    mesh = plsc.VectorSubcoreMesh(core_axis_name="c", subcore_axis_name="s")

    @functools.partial(
        pl.kernel,
        mesh=mesh,
        out_type=jax.ShapeDtypeStruct((TOTAL, D), jnp.float32),
        scratch_types=[
            pltpu.VMEM((C,), jnp.int32),
            pltpu.VMEM((C, D), jnp.float32),
            pltpu.SemaphoreType.DMA,
        ],
    )
    def k(fs_hbm, x_hbm, out_hbm, idx_v, rows_v, sem):
        wid = lax.axis_index("s") * NC + lax.axis_index("c")
        base = NCHUNK // NW
        extra = NCHUNK - base * NW
        n = base + (wid < extra).astype(jnp.int32)

        def body(i, carry):
            c = wid + i * NW
            start = c * C
            bid = c // CPB
            pltpu.sync_copy(fs_hbm.at[pl.ds(start, C)], idx_v)
            pltpu.async_copy(x_hbm.at[bid].at[idx_v], rows_v, sem).wait()
            pltpu.sync_copy(rows_v, out_hbm.at[pl.ds(start, C)])
            return carry

        lax.fori_loop(0, n, body, 0)

    return k(fs_flat, x)


def kernel(fs, x):
    fs_flat = fs.reshape(TOTAL).astype(jnp.int32)
    out = _sc_gather(fs_flat, x)
    return out.reshape(B, F, D)


# speedup vs baseline: 23.3398x; 2.1936x over previous
import functools

import jax
import jax.numpy as jnp
from jax import lax
from jax.experimental import pallas as pl
from jax.experimental.pallas import tpu as pltpu
from jax.experimental.pallas import tpu_sc as plsc

B, F, V, D = 4, 100000, 50000, 128
C = 80
TOTAL = B * F
NCHUNK = TOTAL // C
CPB = F // C
NC, NS = 2, 16
NW = NC * NS
BASEC = NCHUNK // NW
EXTRA = NCHUNK - BASEC * NW
G = 4
ROWS = G * C
NGRP = BASEC // G
IDXN = (BASEC + 1) * C


def _sc_gather(fs_flat, x):
    mesh = plsc.VectorSubcoreMesh(core_axis_name="c", subcore_axis_name="s")

    @functools.partial(
        pl.kernel,
        mesh=mesh,
        out_type=jax.ShapeDtypeStruct((TOTAL, D), jnp.float32),
        scratch_types=[
            pltpu.VMEM((IDXN,), jnp.int32),
            pltpu.VMEM((ROWS, D), jnp.float32),
            pltpu.VMEM((ROWS, D), jnp.float32),
            pltpu.SemaphoreType.DMA,
            pltpu.SemaphoreType.DMA,
        ],
    )
    def k(fs_hbm, x_hbm, out_hbm, idx_v, rows0, rows1, sem0, sem1):
        wid = lax.axis_index("s") * NC + lax.axis_index("c")
        start_chunk = wid * BASEC + jnp.minimum(wid, EXTRA)
        flat_start = start_chunk * C
        has_extra = wid < EXTRA

        pltpu.sync_copy(
            fs_hbm.at[pl.ds(flat_start, BASEC * C)],
            idx_v.at[pl.ds(0, BASEC * C)],
        )

        @pl.when(has_extra)
        def _():
            pltpu.sync_copy(
                fs_hbm.at[pl.ds(flat_start + BASEC * C, C)],
                idx_v.at[pl.ds(BASEC * C, C)],
            )

        def fire(g, rows, sem):
            handles = []
            for b in range(G):
                c = g * G + b
                bid = (start_chunk + c) // CPB
                handles.append(
                    pltpu.async_copy(
                        x_hbm.at[bid].at[idx_v.at[pl.ds(c * C, C)]],
                        rows.at[pl.ds(b * C, C)],
                        sem,
                    )
                )
            return handles

        def drain(handles):
            for h in handles:
                h.wait()

        def outcopy(g, rows):
            pltpu.sync_copy(rows, out_hbm.at[pl.ds(flat_start + g * ROWS, ROWS)])

        def body(t, carry):
            g0 = 2 * t
            h0 = fire(g0, rows0, sem0)

            @pl.when(t > 0)
            def _():
                outcopy(g0 - 1, rows1)

            drain(h0)
            h1 = fire(g0 + 1, rows1, sem1)
            outcopy(g0, rows0)
            drain(h1)
            return carry

        lax.fori_loop(0, (NGRP - 1) // 2, body, 0)

        g_last = NGRP - 1
        h0 = fire(g_last, rows0, sem0)
        outcopy(g_last - 1, rows1)
        drain(h0)

        tail_c = BASEC
        tail_bid = (start_chunk + tail_c) // CPB

        @pl.when(has_extra)
        def _():
            h = pltpu.async_copy(
                x_hbm.at[tail_bid].at[idx_v.at[pl.ds(tail_c * C, C)]],
                rows1.at[pl.ds(0, C)],
                sem1,
            )
            outcopy(g_last, rows0)
            h.wait()
            pltpu.sync_copy(
                rows1.at[pl.ds(0, C)],
                out_hbm.at[pl.ds(flat_start + tail_c * C, C)],
            )

        @pl.when(jnp.logical_not(has_extra))
        def _():
            outcopy(g_last, rows0)

    return k(fs_flat, x)


def kernel(fs, x):
    fs_flat = fs.reshape(TOTAL).astype(jnp.int32)
    out = _sc_gather(fs_flat, x)
    return out.reshape(B, F, D)


# speedup vs baseline: 23.4000x; 1.0026x over previous
import functools

import jax
import jax.numpy as jnp
from jax import lax
from jax.experimental import pallas as pl
from jax.experimental.pallas import tpu as pltpu
from jax.experimental.pallas import tpu_sc as plsc

B, F, V, D = 4, 100000, 50000, 128
C = 80
TOTAL = B * F
NCHUNK = TOTAL // C
CPB = F // C
NC, NS = 2, 16
NW = NC * NS
BASEC = NCHUNK // NW
EXTRA = NCHUNK - BASEC * NW
G = 3
ROWS = G * C
NGRP = BASEC // G
IDXN = (BASEC + 1) * C


def _sc_gather(fs_flat, x):
    mesh = plsc.VectorSubcoreMesh(core_axis_name="c", subcore_axis_name="s")

    @functools.partial(
        pl.kernel,
        mesh=mesh,
        out_type=jax.ShapeDtypeStruct((TOTAL, D), jnp.float32),
        scratch_types=[
            pltpu.VMEM((IDXN,), jnp.int32),
            [pltpu.VMEM((ROWS, D), jnp.float32) for _ in range(3)],
            [pltpu.SemaphoreType.DMA for _ in range(3)],
            [pltpu.SemaphoreType.DMA for _ in range(3)],
        ],
    )
    def k(fs_hbm, x_hbm, out_hbm, idx_v, rows, gsem, osem):
        wid = lax.axis_index("s") * NC + lax.axis_index("c")
        start_chunk = wid * BASEC + jnp.minimum(wid, EXTRA)
        flat_start = start_chunk * C
        has_extra = wid < EXTRA

        pltpu.sync_copy(
            fs_hbm.at[pl.ds(flat_start, BASEC * C)],
            idx_v.at[pl.ds(0, BASEC * C)],
        )

        @pl.when(has_extra)
        def _():
            pltpu.sync_copy(
                fs_hbm.at[pl.ds(flat_start + BASEC * C, C)],
                idx_v.at[pl.ds(BASEC * C, C)],
            )

        def fire(g, s):
            for b in range(G):
                c = g * G + b
                bid = (start_chunk + c) // CPB
                pltpu.async_copy(
                    x_hbm.at[bid].at[idx_v.at[pl.ds(c * C, C)]],
                    rows[s].at[pl.ds(b * C, C)],
                    gsem[s],
                )

        def drain_gathers(s):
            for b in range(G):
                pltpu.make_async_copy(
                    x_hbm.at[0].at[pl.ds(0, C)],
                    rows[s].at[pl.ds(b * C, C)],
                    gsem[s],
                ).wait()

        def out_slice(g):
            return out_hbm.at[pl.ds(flat_start + g * ROWS, ROWS)]

        def fire_out(g, s):
            pltpu.async_copy(rows[s], out_slice(g), osem[s])

        def drain_out(g, s):
            pltpu.make_async_copy(rows[s], out_slice(g), osem[s]).wait()

        def step(g, s, s_prev, first, wait_slot):
            if wait_slot:
                drain_out(g - 3, s)
            fire(g, s)
            if not first:
                drain_gathers(s_prev)
                fire_out(g - 1, s_prev)

        def body(t, carry):
            g0 = 3 * t

            @pl.when(t == 0)
            def _():
                step(0, 0, 2, True, False)
                step(1, 1, 0, False, False)
                step(2, 2, 1, False, False)

            @pl.when(t > 0)
            def _():
                step(g0, 0, 2, False, True)
                step(g0 + 1, 1, 0, False, True)
                step(g0 + 2, 2, 1, False, True)

            return carry

        lax.fori_loop(0, NGRP // 3, body, 0)

        g_last = NGRP - 1
        drain_out(g_last - 3, 0)
        fire(g_last, 0)
        drain_gathers(2)
        fire_out(g_last - 1, 2)
        drain_gathers(0)
        fire_out(g_last, 0)

        tail_c = BASEC
        tail_bid = (start_chunk + tail_c) // CPB
        drain_out(g_last - 2, 1)

        @pl.when(has_extra)
        def _():
            h = pltpu.async_copy(
                x_hbm.at[tail_bid].at[idx_v.at[pl.ds(tail_c * C, C)]],
                rows[1].at[pl.ds(0, C)],
                gsem[1],
            )
            h.wait()
            pltpu.sync_copy(
                rows[1].at[pl.ds(0, C)],
                out_hbm.at[pl.ds(flat_start + tail_c * C, C)],
            )

        drain_out(g_last - 1, 2)
        drain_out(g_last, 0)

    return k(fs_flat, x)


def kernel(fs, x):
    fs_flat = fs.reshape(TOTAL).astype(jnp.int32)
    out = _sc_gather(fs_flat, x)
    return out.reshape(B, F, D)


# speedup vs baseline: 23.7841x; 1.0164x over previous
import functools

import jax
import jax.numpy as jnp
from jax import lax
from jax.experimental import pallas as pl
from jax.experimental.pallas import tpu as pltpu
from jax.experimental.pallas import tpu_sc as plsc

B, F, V, D = 4, 100000, 50000, 128
TOTAL = B * F
NC, NS = 2, 16
NW = NC * NS
WPB = NW // B
PW = F // WPB
PWE = PW + 4
PWO = PW - 4
C = 128
NCH = PWO // C
TAILE = PWE - NCH * C
TAILO = PWO - NCH * C


def _sc_gather(fs_flat, x):
    mesh = plsc.VectorSubcoreMesh(core_axis_name="c", subcore_axis_name="s")

    @functools.partial(
        pl.kernel,
        mesh=mesh,
        out_type=jax.ShapeDtypeStruct((TOTAL, D), jnp.float32),
        scratch_types=[
            pltpu.VMEM((PWE,), jnp.int32),
            [pltpu.VMEM((C, D), jnp.float32) for _ in range(3)],
            [pltpu.SemaphoreType.DMA for _ in range(3)],
            [pltpu.SemaphoreType.DMA for _ in range(3)],
        ],
    )
    def k(fs_hbm, x_hbm, out_hbm, idx_v, rows, gsem, osem):
        wid = lax.axis_index("s") * NC + lax.axis_index("c")
        bid = wid // WPB
        j = wid % WPB
        par = j % 2
        start = pl.multiple_of(bid * F + j * PW + 4 * par, 8)
        even = par == 0

        pltpu.sync_copy(fs_hbm.at[pl.ds(start, PWO)], idx_v.at[pl.ds(0, PWO)])

        @pl.when(even)
        def _():
            pltpu.sync_copy(
                fs_hbm.at[pl.ds(start + PWO, 8)], idx_v.at[pl.ds(PWO, 8)]
            )

        def fire(c, s, n):
            pltpu.async_copy(
                x_hbm.at[bid].at[idx_v.at[pl.ds(c * C, n)]],
                rows[s].at[pl.ds(0, n)],
                gsem[s],
            )

        def drain_gather(s, n):
            pltpu.make_async_copy(
                x_hbm.at[0].at[pl.ds(0, n)],
                rows[s].at[pl.ds(0, n)],
                gsem[s],
            ).wait()

        def out_slice(c, n):
            return out_hbm.at[pl.ds(start + c * C, n)]

        def fire_out(c, s, n):
            pltpu.async_copy(rows[s].at[pl.ds(0, n)], out_slice(c, n), osem[s])

        def drain_out(c, s, n):
            pltpu.make_async_copy(
                rows[s].at[pl.ds(0, n)], out_slice(c, n), osem[s]
            ).wait()

        def step(c, s, s_prev, first, wait_slot):
            if wait_slot:
                drain_out(c - 3, s, C)
            fire(c, s, C)
            if not first:
                drain_gather(s_prev, C)
                fire_out(c - 1, s_prev, C)

        def body(t, carry):
            c0 = 3 * t

            @pl.when(t == 0)
            def _():
                step(0, 0, 2, True, False)
                step(1, 1, 0, False, False)
                step(2, 2, 1, False, False)

            @pl.when(t > 0)
            def _():
                step(c0, 0, 2, False, True)
                step(c0 + 1, 1, 0, False, True)
                step(c0 + 2, 2, 1, False, True)

            return carry

        lax.fori_loop(0, 32, body, 0)

        drain_out(93, 0, C)
        fire(96, 0, C)
        drain_gather(2, C)
        fire_out(95, 2, C)

        drain_out(94, 1, C)

        def tail(n):
            fire(NCH, 1, n)
            drain_gather(0, C)
            fire_out(96, 0, C)
            drain_gather(1, n)
            fire_out(NCH, 1, n)
            drain_out(95, 2, C)
            drain_out(96, 0, C)
            drain_out(NCH, 1, n)

        @pl.when(even)
        def _():
            tail(TAILE)

        @pl.when(jnp.logical_not(even))
        def _():
            tail(TAILO)

    return k(fs_flat, x)


def kernel(fs, x):
    fs_flat = fs.reshape(TOTAL).astype(jnp.int32)
    out = _sc_gather(fs_flat, x)
    return out.reshape(B, F, D)
